# Initial kernel scaffold; baseline (speedup 1.0000x reference)
import jax
import jax.numpy as jnp
from jax.experimental import pallas as pl


def kernel(x, edge_index, W1, b1, g1, be1, W2, b2, g2, be2, W3, b3):
    raise NotImplementedError("write your pallas kernel here")


# speedup vs baseline: 13.9896x; 13.9896x over previous
import functools

import jax
import jax.numpy as jnp
from jax import lax
from jax.experimental import pallas as pl
from jax.experimental.pallas import tpu as pltpu
from jax.experimental.pallas import tpu_sc as plsc

_NC = 2
_NS = 16
_NT = _NC * _NS
_CH = 80


def _sc_deg(dstr, n):
  ng = dstr.shape[1]
  mesh = plsc.VectorSubcoreMesh(core_axis_name="c", subcore_axis_name="s")

  @functools.partial(
      pl.kernel,
      out_type=jax.ShapeDtypeStruct((_NT, 1, n), jnp.float32),
      mesh=mesh,
      scratch_types=[
          pltpu.VMEM((ng, 16), jnp.int32),
          pltpu.VMEM((n,), jnp.float32),
      ],
      compiler_params=pltpu.CompilerParams(needs_layout_passes=False),
  )
  def k(dstr_hbm, degp_hbm, dst_v, hist):
    c = lax.axis_index("c")
    s = lax.axis_index("s")
    wid = c * _NS + s
    pltpu.sync_copy(dstr_hbm.at[wid], dst_v)
    zer = jnp.zeros((16,), jnp.float32)
    one = jnp.ones((16,), jnp.float32)

    def fill_z(i, carry):
      hist[pl.ds(i * 16, 16)] = zer
      return carry

    lax.fori_loop(0, n // 16, fill_z, 0)

    def group(g, carry):
      plsc.addupdate_scatter(hist, [dst_v[g, :]], one)
      return carry

    lax.fori_loop(0, ng, group, 0)
    pltpu.sync_copy(hist, degp_hbm.at[wid, 0])

  return k(dstr).reshape(_NT, n)


def _sc_conv_fs(hwp_flat, srcr2, dstr, n):
  nch = srcr2.shape[2]
  npt = n // _NS
  mesh = plsc.VectorSubcoreMesh(core_axis_name="c", subcore_axis_name="s")

  @functools.partial(
      pl.kernel,
      out_type=jax.ShapeDtypeStruct((_NC, _NS, npt, 64), jnp.float32),
      mesh=mesh,
      scratch_types=[
          pltpu.VMEM((nch, _CH), jnp.int32),
          pltpu.VMEM((nch, _CH), jnp.int32),
          pltpu.VMEM((_CH, 64), jnp.float32),
          pltpu.VMEM((125, 64), jnp.float32),
          pltpu.VMEM_SHARED((n, 64), jnp.float32),
          pltpu.SemaphoreType.DMA,
      ],
      compiler_params=pltpu.CompilerParams(needs_layout_passes=False,
                                           use_tc_tiling_on_sc=False),
  )
  def k(hwp_hbm, srcr_hbm, dstr_hbm, agg_hbm,
        src_v, dst_v, rowbuf, zbuf, agg_sh, sem):
    c = lax.axis_index("c")
    s = lax.axis_index("s")
    pltpu.sync_copy(srcr_hbm.at[c, s], src_v)
    pltpu.sync_copy(dstr_hbm.at[s], dst_v)
    zer = jnp.zeros((16,), jnp.float32)

    def fill_z(i, carry):
      for j in range(4):
        zbuf[i, pl.ds(j * 16, 16)] = zer
      return carry

    lax.fori_loop(0, 125, fill_z, 0)
    for r in range(npt // 125):
      pltpu.sync_copy(zbuf, agg_sh.at[pl.ds(s * npt + r * 125, 125)])
    plsc.subcore_barrier()

    def chunk(g, carry):
      pltpu.async_copy(hwp_hbm.at[src_v.at[g]], rowbuf, sem).wait()
      pltpu.sync_copy(rowbuf, agg_sh.at[dst_v.at[g]], add=True)
      return carry

    lax.fori_loop(0, nch, chunk, 0)
    plsc.subcore_barrier()
    pltpu.sync_copy(agg_sh.at[pl.ds(s * npt, npt)], agg_hbm.at[c, s])

  return k(hwp_flat, srcr2, dstr).reshape(_NC, n, 64)


def _sc_conv_es(hwp, srcr, dstr, n):
  nch = srcr.shape[1]
  npt = n // _NS
  mesh = plsc.VectorSubcoreMesh(core_axis_name="c", subcore_axis_name="s")

  @functools.partial(
      pl.kernel,
      out_type=jax.ShapeDtypeStruct((_NC, _NS, npt, 64), jnp.float32),
      mesh=mesh,
      scratch_types=[
          pltpu.VMEM((nch, _CH), jnp.int32),
          pltpu.VMEM((nch, _CH), jnp.int32),
          pltpu.VMEM((_CH, 64), jnp.float32),
          pltpu.VMEM((125, 64), jnp.float32),
          pltpu.VMEM_SHARED((n, 64), jnp.float32),
          pltpu.SemaphoreType.DMA,
      ],
      compiler_params=pltpu.CompilerParams(needs_layout_passes=False,
                                           use_tc_tiling_on_sc=False),
  )
  def k(hwp_hbm, srcr_hbm, dstr_hbm, agg_hbm,
        src_v, dst_v, rowbuf, zbuf, agg_sh, sem):
    c = lax.axis_index("c")
    s = lax.axis_index("s")
    wid = c * _NS + s
    pltpu.sync_copy(srcr_hbm.at[wid], src_v)
    pltpu.sync_copy(dstr_hbm.at[wid], dst_v)
    zer = jnp.zeros((16,), jnp.float32)

    def fill_z(i, carry):
      for j in range(4):
        zbuf[i, pl.ds(j * 16, 16)] = zer
      return carry

    lax.fori_loop(0, 125, fill_z, 0)
    for r in range(npt // 125):
      pltpu.sync_copy(zbuf, agg_sh.at[pl.ds(s * npt + r * 125, 125)])
    plsc.subcore_barrier()

    def chunk(g, carry):
      pltpu.async_copy(hwp_hbm.at[src_v.at[g]], rowbuf, sem).wait()
      pltpu.sync_copy(rowbuf, agg_sh.at[dst_v.at[g]], add=True)
      return carry

    lax.fori_loop(0, nch, chunk, 0)
    plsc.subcore_barrier()
    pltpu.sync_copy(agg_sh.at[pl.ds(s * npt, npt)], agg_hbm.at[c, s])

  return k(hwp, srcr, dstr).reshape(_NC, n, 64)


def _tc_head(degp_t, x, w1):
  n = x.shape[0]

  def body(degp_ref, x_ref, w_ref, hwp_ref, dis_ref):
    deg = jnp.sum(degp_ref[...], axis=1, keepdims=True) + 1.0
    dis = lax.rsqrt(deg)
    hw = jnp.dot(x_ref[...], w_ref[...],
                 preferred_element_type=jnp.float32) * dis
    hwp_ref[0] = hw[:, 0:64]
    hwp_ref[1] = hw[:, 64:128]
    dis_ref[...] = dis

  return pl.pallas_call(
      body,
      out_shape=(jax.ShapeDtypeStruct((_NC, n, 64), jnp.float32),
                 jax.ShapeDtypeStruct((n, 1), jnp.float32)),
  )(degp_t, x, w1)


def _tc_tail(agg_in, hwp, dis, b, g, be, wn, split_out):
  n = hwp.shape[1]

  def body(agg_ref, hwp_ref, dis_ref, b_ref, g_ref, be_ref, w_ref, out_ref):
    dis = dis_ref[...]
    agg = (agg_ref[...] + hwp_ref[...]) * dis[None] + b_ref[...]
    mu = jnp.mean(agg, axis=1, keepdims=True)
    var = jnp.mean((agg - mu) ** 2, axis=1, keepdims=True)
    h3 = jax.nn.relu((agg - mu) * lax.rsqrt(var + 1e-5) * g_ref[...]
                     + be_ref[...])
    h = jnp.concatenate([h3[0], h3[1]], axis=1)
    hw = jnp.dot(h, w_ref[...], preferred_element_type=jnp.float32) * dis
    if split_out:
      out_ref[0] = hw[:, 0:64]
      out_ref[1] = hw[:, 64:128]
    else:
      out_ref[...] = hw

  out_shape = (jax.ShapeDtypeStruct((_NC, n, 64), jnp.float32) if split_out
               else jax.ShapeDtypeStruct((n, wn.shape[1]), jnp.float32))
  return pl.pallas_call(body, out_shape=out_shape)(
      agg_in, hwp, dis, b.reshape(_NC, 1, 64), g.reshape(_NC, 1, 64),
      be.reshape(_NC, 1, 64), wn)


def _tc_final(aggp, hwp, dis, b3, d_out):
  n = hwp.shape[0]

  def body(aggp_ref, hwp_ref, dis_ref, b_ref, out_ref):
    tot = (aggp_ref[0] + aggp_ref[1] + hwp_ref[...]) * dis_ref[...]
    z = tot[:, 0:d_out] + b_ref[...]
    m = jnp.max(z, axis=1, keepdims=True)
    e = jnp.exp(z - m)
    out_ref[...] = e / jnp.sum(e, axis=1, keepdims=True)

  return pl.pallas_call(
      body,
      out_shape=jax.ShapeDtypeStruct((n, d_out), jnp.float32),
  )(aggp, hwp, dis, b3.reshape(1, -1))


def kernel(x, edge_index, W1, b1, g1, be1, W2, b2, g2, be2, W3, b3):
  n = x.shape[0]
  e = edge_index.shape[1]
  d_out = W3.shape[1]
  assert e % (_NT * _CH) == 0 and n % (_NS * 125) == 0 and n % 16 == 0

  src = edge_index[0]
  dst = edge_index[1]
  srcr_fs = src.reshape(_NS, e // (_NS * _CH), _CH)
  srcr2 = jnp.stack([srcr_fs, srcr_fs + n])
  dstr_fs = dst.reshape(_NS, e // (_NS * _CH), _CH)
  srcr_es = src.reshape(_NT, e // (_NT * _CH), _CH)
  dstr_es = dst.reshape(_NT, e // (_NT * _CH), _CH)
  dstr_d = dst.reshape(_NT, e // (_NT * 16), 16)

  degp = _sc_deg(dstr_d, n)
  hw1p, dis = _tc_head(degp.T, x, W1)
  agg1 = _sc_conv_fs(hw1p.reshape(_NC * n, 64), srcr2, dstr_fs, n)
  hw2p = _tc_tail(agg1, hw1p, dis, b1, g1, be1, W2, True)
  agg2 = _sc_conv_fs(hw2p.reshape(_NC * n, 64), srcr2, dstr_fs, n)
  w3p = jnp.pad(W3, ((0, 0), (0, 64 - d_out)))
  hw3p = _tc_tail(agg2, hw2p, dis, b2, g2, be2, w3p, False)
  agg3p = _sc_conv_es(hw3p, srcr_es, dstr_es, n)
  return _tc_final(agg3p, hw3p, dis, b3, d_out)


# speedup vs baseline: 27.0473x; 1.9334x over previous
import functools

import jax
import jax.numpy as jnp
from jax import lax
from jax.experimental import pallas as pl
from jax.experimental.pallas import tpu as pltpu
from jax.experimental.pallas import tpu_sc as plsc

_NC = 2
_NS = 16
_NT = _NC * _NS
_CH = 80
_NBUF = 5


def _sc_deg(dstr, n):
  ng = dstr.shape[1]
  mesh = plsc.VectorSubcoreMesh(core_axis_name="c", subcore_axis_name="s")

  @functools.partial(
      pl.kernel,
      out_type=jax.ShapeDtypeStruct((_NT, 1, n), jnp.float32),
      mesh=mesh,
      scratch_types=[
          pltpu.VMEM((ng, 16), jnp.int32),
          pltpu.VMEM((n,), jnp.float32),
      ],
      compiler_params=pltpu.CompilerParams(needs_layout_passes=False),
  )
  def k(dstr_hbm, degp_hbm, dst_v, hist):
    c = lax.axis_index("c")
    s = lax.axis_index("s")
    wid = c * _NS + s
    pltpu.sync_copy(dstr_hbm.at[wid], dst_v)
    zer = jnp.zeros((16,), jnp.float32)
    one = jnp.ones((16,), jnp.float32)

    def fill_z(i, carry):
      hist[pl.ds(i * 16, 16)] = zer
      return carry

    lax.fori_loop(0, n // 16, fill_z, 0)

    def group(g, carry):
      plsc.addupdate_scatter(hist, [dst_v[g, :]], one)
      return carry

    lax.fori_loop(0, ng, group, 0)
    pltpu.sync_copy(hist, degp_hbm.at[wid, 0])

  return k(dstr).reshape(_NT, n)


def _sc_conv(hwp_tab, srcr, dstr, n, feature_split):
  nch = srcr.shape[-2]
  npt = n // _NS
  nrnd = nch // _NBUF
  assert nch % _NBUF == 0
  mesh = plsc.VectorSubcoreMesh(core_axis_name="c", subcore_axis_name="s")

  @functools.partial(
      pl.kernel,
      out_type=jax.ShapeDtypeStruct((_NC, _NS, npt, 64), jnp.float32),
      mesh=mesh,
      scratch_types=[
          pltpu.VMEM((nch, _CH), jnp.int32),
          pltpu.VMEM((nch, _CH), jnp.int32),
          pltpu.VMEM((_NBUF, _CH, 64), jnp.float32),
          pltpu.VMEM((125, 64), jnp.float32),
          pltpu.VMEM_SHARED((n, 64), jnp.float32),
      ] + [pltpu.SemaphoreType.DMA] * (2 * _NBUF),
      compiler_params=pltpu.CompilerParams(needs_layout_passes=False,
                                           use_tc_tiling_on_sc=False),
  )
  def k(hwp_hbm, srcr_hbm, dstr_hbm, agg_hbm,
        src_v, dst_v, bufs, zbuf, agg_sh, *sems):
    gsem = sems[:_NBUF]
    ssem = sems[_NBUF:]
    c = lax.axis_index("c")
    s = lax.axis_index("s")
    if feature_split:
      pltpu.sync_copy(srcr_hbm.at[c, s], src_v)
      pltpu.sync_copy(dstr_hbm.at[s], dst_v)
    else:
      wid = c * _NS + s
      pltpu.sync_copy(srcr_hbm.at[wid], src_v)
      pltpu.sync_copy(dstr_hbm.at[wid], dst_v)
    zer = jnp.zeros((16,), jnp.float32)

    def fill_z(i, carry):
      for j in range(4):
        zbuf[i, pl.ds(j * 16, 16)] = zer
      return carry

    lax.fori_loop(0, 125, fill_z, 0)
    for r in range(npt // 125):
      pltpu.sync_copy(zbuf, agg_sh.at[pl.ds(s * npt + r * 125, 125)])
    plsc.subcore_barrier()

    for b in range(_NBUF):
      pltpu.async_copy(hwp_hbm.at[src_v.at[b]], bufs.at[b], gsem[b])

    def rnd(t, carry):
      g0 = t * _NBUF
      for b in range(_NBUF):
        g = g0 + b
        pltpu.make_async_copy(hwp_hbm.at[src_v.at[g]], bufs.at[b],
                              gsem[b]).wait()
        pltpu.async_copy(bufs.at[b], agg_sh.at[dst_v.at[g]], ssem[b],
                         add=True)
      for b in range(_NBUF):
        g = g0 + b
        nxt = jnp.minimum(g + _NBUF, nch - 1)
        pltpu.make_async_copy(bufs.at[b], agg_sh.at[dst_v.at[g]],
                              ssem[b]).wait()
        pltpu.async_copy(hwp_hbm.at[src_v.at[nxt]], bufs.at[b], gsem[b])
      return carry

    lax.fori_loop(0, nrnd, rnd, 0)
    for b in range(_NBUF):
      pltpu.make_async_copy(hwp_hbm.at[src_v.at[0]], bufs.at[b],
                            gsem[b]).wait()
    plsc.subcore_barrier()
    pltpu.sync_copy(agg_sh.at[pl.ds(s * npt, npt)], agg_hbm.at[c, s])

  return k(hwp_tab, srcr, dstr).reshape(_NC, n, 64)


def _tc_head(degp_t, x, w1):
  n = x.shape[0]

  def body(degp_ref, x_ref, w_ref, hwp_ref, dis_ref):
    deg = jnp.sum(degp_ref[...], axis=1, keepdims=True) + 1.0
    dis = lax.rsqrt(deg)
    hw = jnp.dot(x_ref[...], w_ref[...],
                 preferred_element_type=jnp.float32) * dis
    hwp_ref[0] = hw[:, 0:64]
    hwp_ref[1] = hw[:, 64:128]
    dis_ref[...] = dis

  return pl.pallas_call(
      body,
      out_shape=(jax.ShapeDtypeStruct((_NC, n, 64), jnp.float32),
                 jax.ShapeDtypeStruct((n, 1), jnp.float32)),
  )(degp_t, x, w1)


def _tc_tail(agg_in, hwp, dis, b, g, be, wn, split_out):
  n = hwp.shape[1]

  def body(agg_ref, hwp_ref, dis_ref, b_ref, g_ref, be_ref, w_ref, out_ref):
    dis = dis_ref[...]
    agg = (agg_ref[...] + hwp_ref[...]) * dis[None] + b_ref[...]
    mu = jnp.mean(agg, axis=1, keepdims=True)
    var = jnp.mean((agg - mu) ** 2, axis=1, keepdims=True)
    h3 = jax.nn.relu((agg - mu) * lax.rsqrt(var + 1e-5) * g_ref[...]
                     + be_ref[...])
    h = jnp.concatenate([h3[0], h3[1]], axis=1)
    hw = jnp.dot(h, w_ref[...], preferred_element_type=jnp.float32) * dis
    if split_out:
      out_ref[0] = hw[:, 0:64]
      out_ref[1] = hw[:, 64:128]
    else:
      out_ref[...] = hw

  out_shape = (jax.ShapeDtypeStruct((_NC, n, 64), jnp.float32) if split_out
               else jax.ShapeDtypeStruct((n, wn.shape[1]), jnp.float32))
  return pl.pallas_call(body, out_shape=out_shape)(
      agg_in, hwp, dis, b.reshape(_NC, 1, 64), g.reshape(_NC, 1, 64),
      be.reshape(_NC, 1, 64), wn)


def _tc_final(aggp, hwp, dis, b3, d_out):
  n = hwp.shape[0]

  def body(aggp_ref, hwp_ref, dis_ref, b_ref, out_ref):
    tot = (aggp_ref[0] + aggp_ref[1] + hwp_ref[...]) * dis_ref[...]
    z = tot[:, 0:d_out] + b_ref[...]
    m = jnp.max(z, axis=1, keepdims=True)
    e = jnp.exp(z - m)
    out_ref[...] = e / jnp.sum(e, axis=1, keepdims=True)

  return pl.pallas_call(
      body,
      out_shape=jax.ShapeDtypeStruct((n, d_out), jnp.float32),
  )(aggp, hwp, dis, b3.reshape(1, -1))


def kernel(x, edge_index, W1, b1, g1, be1, W2, b2, g2, be2, W3, b3):
  n = x.shape[0]
  e = edge_index.shape[1]
  d_out = W3.shape[1]
  assert e % (_NT * _CH) == 0 and n % (_NS * 125) == 0 and n % 16 == 0

  src = edge_index[0]
  dst = edge_index[1]
  srcr_fs = src.reshape(_NS, e // (_NS * _CH), _CH)
  srcr2 = jnp.stack([srcr_fs, srcr_fs + n])
  dstr_fs = dst.reshape(_NS, e // (_NS * _CH), _CH)
  srcr_es = src.reshape(_NT, e // (_NT * _CH), _CH)
  dstr_es = dst.reshape(_NT, e // (_NT * _CH), _CH)
  dstr_d = dst.reshape(_NT, e // (_NT * 16), 16)

  degp = _sc_deg(dstr_d, n)
  hw1p, dis = _tc_head(degp.T, x, W1)
  agg1 = _sc_conv(hw1p.reshape(_NC * n, 64), srcr2, dstr_fs, n, True)
  hw2p = _tc_tail(agg1, hw1p, dis, b1, g1, be1, W2, True)
  agg2 = _sc_conv(hw2p.reshape(_NC * n, 64), srcr2, dstr_fs, n, True)
  w3p = jnp.pad(W3, ((0, 0), (0, 64 - d_out)))
  hw3p = _tc_tail(agg2, hw2p, dis, b2, g2, be2, w3p, False)
  agg3p = _sc_conv(hw3p, srcr_es, dstr_es, n, False)
  return _tc_final(agg3p, hw3p, dis, b3, d_out)


# speedup vs baseline: 27.4011x; 1.0131x over previous
import functools

import jax
import jax.numpy as jnp
from jax import lax
from jax.experimental import pallas as pl
from jax.experimental.pallas import tpu as pltpu
from jax.experimental.pallas import tpu_sc as plsc

_NC = 2
_NS = 16
_NT = _NC * _NS
_CH = 80
_NBUF = 5


def _sc_deg(dstr, n):
  ng = dstr.shape[1]
  mesh = plsc.VectorSubcoreMesh(core_axis_name="c", subcore_axis_name="s")

  @functools.partial(
      pl.kernel,
      out_type=jax.ShapeDtypeStruct((_NT, 1, n), jnp.float32),
      mesh=mesh,
      scratch_types=[
          pltpu.VMEM((ng, 16), jnp.int32),
          pltpu.VMEM((n,), jnp.float32),
      ],
      compiler_params=pltpu.CompilerParams(needs_layout_passes=False),
  )
  def k(dstr_hbm, degp_hbm, dst_v, hist):
    c = lax.axis_index("c")
    s = lax.axis_index("s")
    wid = c * _NS + s
    pltpu.sync_copy(dstr_hbm.at[wid], dst_v)
    zer = jnp.zeros((16,), jnp.float32)
    one = jnp.ones((16,), jnp.float32)

    def fill_z(i, carry):
      hist[pl.ds(i * 16, 16)] = zer
      return carry

    lax.fori_loop(0, n // 16, fill_z, 0)

    def group(g, carry):
      plsc.addupdate_scatter(hist, [dst_v[g, :]], one)
      return carry

    lax.fori_loop(0, ng, group, 0)
    pltpu.sync_copy(hist, degp_hbm.at[wid, 0])

  return k(dstr).reshape(_NT, n)


def _sc_conv(hwp_tab, srcr, dstr, n, feature_split):
  nch = srcr.shape[-2]
  npt = n // _NS
  nrnd = nch // _NBUF
  assert nch % _NBUF == 0
  mesh = plsc.VectorSubcoreMesh(core_axis_name="c", subcore_axis_name="s")

  @functools.partial(
      pl.kernel,
      out_type=jax.ShapeDtypeStruct((_NC, _NS, npt, 64), jnp.float32),
      mesh=mesh,
      scratch_types=[
          pltpu.VMEM((nch, _CH), jnp.int32),
          pltpu.VMEM((nch, _CH), jnp.int32),
          pltpu.VMEM((_NBUF, _CH, 64), jnp.float32),
          pltpu.VMEM((125, 64), jnp.float32),
          pltpu.VMEM_SHARED((n, 64), jnp.float32),
      ] + [pltpu.SemaphoreType.DMA] * (2 * _NBUF),
      compiler_params=pltpu.CompilerParams(needs_layout_passes=False,
                                           use_tc_tiling_on_sc=False),
  )
  def k(hwp_hbm, srcr_hbm, dstr_hbm, agg_hbm,
        src_v, dst_v, bufs, zbuf, agg_sh, *sems):
    gsem = sems[:_NBUF]
    ssem = sems[_NBUF:]
    c = lax.axis_index("c")
    s = lax.axis_index("s")
    if feature_split:
      pltpu.sync_copy(srcr_hbm.at[s], src_v)
      pltpu.sync_copy(dstr_hbm.at[s], dst_v)
    else:
      wid = c * _NS + s
      pltpu.sync_copy(srcr_hbm.at[wid], src_v)
      pltpu.sync_copy(dstr_hbm.at[wid], dst_v)
    zer = jnp.zeros((16,), jnp.float32)

    def fill_z(i, carry):
      for j in range(4):
        zbuf[i, pl.ds(j * 16, 16)] = zer
      return carry

    lax.fori_loop(0, 125, fill_z, 0)
    for r in range(npt // 125):
      pltpu.sync_copy(zbuf, agg_sh.at[pl.ds(s * npt + r * 125, 125)])
    plsc.subcore_barrier()

    tab = hwp_hbm.at[c] if feature_split else hwp_hbm
    for b in range(_NBUF):
      pltpu.async_copy(tab.at[src_v.at[b]], bufs.at[b], gsem[b])

    def rnd(t, carry):
      g0 = t * _NBUF
      for b in range(_NBUF):
        g = g0 + b
        pltpu.make_async_copy(tab.at[src_v.at[g]], bufs.at[b],
                              gsem[b]).wait()
        pltpu.async_copy(bufs.at[b], agg_sh.at[dst_v.at[g]], ssem[b],
                         add=True)
      for b in range(_NBUF):
        g = g0 + b
        nxt = jnp.minimum(g + _NBUF, nch - 1)
        pltpu.make_async_copy(bufs.at[b], agg_sh.at[dst_v.at[g]],
                              ssem[b]).wait()
        pltpu.async_copy(tab.at[src_v.at[nxt]], bufs.at[b], gsem[b])
      return carry

    lax.fori_loop(0, nrnd, rnd, 0)
    for b in range(_NBUF):
      pltpu.make_async_copy(tab.at[src_v.at[0]], bufs.at[b],
                            gsem[b]).wait()
    plsc.subcore_barrier()
    pltpu.sync_copy(agg_sh.at[pl.ds(s * npt, npt)], agg_hbm.at[c, s])

  return k(hwp_tab, srcr, dstr).reshape(_NC, n, 64)


def _tc_head(degp_t, x, w1):
  n = x.shape[0]

  def body(degp_ref, x_ref, w_ref, hwp_ref, dis_ref):
    ones = jnp.ones((degp_ref.shape[0], 1), jnp.float32)
    deg = lax.dot_general(degp_ref[...], ones, (((0,), (0,)), ((), ())),
                          preferred_element_type=jnp.float32) + 1.0
    dis = lax.rsqrt(deg)
    hw = jnp.dot(x_ref[...], w_ref[...],
                 preferred_element_type=jnp.float32) * dis
    hwp_ref[0] = hw[:, 0:64]
    hwp_ref[1] = hw[:, 64:128]
    dis_ref[...] = dis

  return pl.pallas_call(
      body,
      out_shape=(jax.ShapeDtypeStruct((_NC, n, 64), jnp.float32),
                 jax.ShapeDtypeStruct((n, 1), jnp.float32)),
  )(degp_t, x, w1)


def _tc_tail(agg_in, hwp, dis, b, g, be, wn, split_out):
  n = hwp.shape[1]

  def body(agg_ref, hwp_ref, dis_ref, b_ref, g_ref, be_ref, w_ref, out_ref):
    dis = dis_ref[...]
    agg = (agg_ref[...] + hwp_ref[...]) * dis[None] + b_ref[...]
    mu = jnp.mean(agg, axis=1, keepdims=True)
    var = jnp.mean((agg - mu) ** 2, axis=1, keepdims=True)
    h3 = jax.nn.relu((agg - mu) * lax.rsqrt(var + 1e-5) * g_ref[...]
                     + be_ref[...])
    h = jnp.concatenate([h3[0], h3[1]], axis=1)
    hw = jnp.dot(h, w_ref[...], preferred_element_type=jnp.float32) * dis
    if split_out:
      out_ref[0] = hw[:, 0:64]
      out_ref[1] = hw[:, 64:128]
    else:
      out_ref[...] = hw

  out_shape = (jax.ShapeDtypeStruct((_NC, n, 64), jnp.float32) if split_out
               else jax.ShapeDtypeStruct((n, wn.shape[1]), jnp.float32))
  return pl.pallas_call(body, out_shape=out_shape)(
      agg_in, hwp, dis, b.reshape(_NC, 1, 64), g.reshape(_NC, 1, 64),
      be.reshape(_NC, 1, 64), wn)


def _tc_final(aggp, hwp, dis, b3, d_out):
  n = hwp.shape[0]

  def body(aggp_ref, hwp_ref, dis_ref, b_ref, out_ref):
    tot = (aggp_ref[0] + aggp_ref[1] + hwp_ref[...]) * dis_ref[...]
    z = tot[:, 0:d_out] + b_ref[...]
    m = jnp.max(z, axis=1, keepdims=True)
    e = jnp.exp(z - m)
    out_ref[...] = e / jnp.sum(e, axis=1, keepdims=True)

  return pl.pallas_call(
      body,
      out_shape=jax.ShapeDtypeStruct((n, d_out), jnp.float32),
  )(aggp, hwp, dis, b3.reshape(1, -1))


def kernel(x, edge_index, W1, b1, g1, be1, W2, b2, g2, be2, W3, b3):
  n = x.shape[0]
  e = edge_index.shape[1]
  d_out = W3.shape[1]
  assert e % (_NT * _CH) == 0 and n % (_NS * 125) == 0 and n % 16 == 0

  src = edge_index[0]
  dst = edge_index[1]
  srcr_fs = src.reshape(_NS, e // (_NS * _CH), _CH)
  dstr_fs = dst.reshape(_NS, e // (_NS * _CH), _CH)
  srcr_es = src.reshape(_NT, e // (_NT * _CH), _CH)
  dstr_es = dst.reshape(_NT, e // (_NT * _CH), _CH)
  dstr_d = dst.reshape(_NT, e // (_NT * 16), 16)

  degp = _sc_deg(dstr_d, n)
  hw1p, dis = _tc_head(degp, x, W1)
  agg1 = _sc_conv(hw1p, srcr_fs, dstr_fs, n, True)
  hw2p = _tc_tail(agg1, hw1p, dis, b1, g1, be1, W2, True)
  agg2 = _sc_conv(hw2p, srcr_fs, dstr_fs, n, True)
  w3p = jnp.pad(W3, ((0, 0), (0, 64 - d_out)))
  hw3p = _tc_tail(agg2, hw2p, dis, b2, g2, be2, w3p, False)
  agg3p = _sc_conv(hw3p, srcr_es, dstr_es, n, False)
  return _tc_final(agg3p, hw3p, dis, b3, d_out)


# speedup vs baseline: 28.0512x; 1.0237x over previous
import functools

import jax
import jax.numpy as jnp
from jax import lax
from jax.experimental import pallas as pl
from jax.experimental.pallas import tpu as pltpu
from jax.experimental.pallas import tpu_sc as plsc

_NC = 2
_NS = 16
_NT = _NC * _NS
_CH = 80
_NBUF = 5


def _sc_deg(dstr, n):
  ng = dstr.shape[1]
  mesh = plsc.VectorSubcoreMesh(core_axis_name="c", subcore_axis_name="s")

  @functools.partial(
      pl.kernel,
      out_type=jax.ShapeDtypeStruct((_NT, 1, n), jnp.float32),
      mesh=mesh,
      scratch_types=[
          pltpu.VMEM((ng, 16), jnp.int32),
          pltpu.VMEM((n,), jnp.float32),
      ],
      compiler_params=pltpu.CompilerParams(needs_layout_passes=False),
  )
  def k(dstr_hbm, degp_hbm, dst_v, hist):
    c = lax.axis_index("c")
    s = lax.axis_index("s")
    wid = c * _NS + s
    pltpu.sync_copy(dstr_hbm.at[wid], dst_v)
    zer = jnp.zeros((16,), jnp.float32)
    one = jnp.ones((16,), jnp.float32)

    def fill_z(i, carry):
      hist[pl.ds(i * 16, 16)] = zer
      return carry

    lax.fori_loop(0, n // 16, fill_z, 0)

    def group(g, carry):
      plsc.addupdate_scatter(hist, [dst_v[g, :]], one)
      return carry

    lax.fori_loop(0, ng, group, 0)
    pltpu.sync_copy(hist, degp_hbm.at[wid, 0])

  return k(dstr).reshape(_NT, n)


def _sc_conv(hwp_tab, srcr, dstr, n, feature_split, width=64, nbuf=_NBUF):
  nch = srcr.shape[-2]
  npt = n // _NS
  nrnd = nch // nbuf
  assert nch % nbuf == 0 and width % 16 == 0
  mesh = plsc.VectorSubcoreMesh(core_axis_name="c", subcore_axis_name="s")

  @functools.partial(
      pl.kernel,
      out_type=jax.ShapeDtypeStruct((_NC, _NS, npt, width), jnp.float32),
      mesh=mesh,
      scratch_types=[
          pltpu.VMEM((nch, _CH), jnp.int32),
          pltpu.VMEM((nch, _CH), jnp.int32),
          pltpu.VMEM((nbuf, _CH, width), jnp.float32),
          pltpu.VMEM((125, width), jnp.float32),
          pltpu.VMEM_SHARED((n, width), jnp.float32),
      ] + [pltpu.SemaphoreType.DMA] * (2 * nbuf),
      compiler_params=pltpu.CompilerParams(needs_layout_passes=False,
                                           use_tc_tiling_on_sc=False),
  )
  def k(hwp_hbm, srcr_hbm, dstr_hbm, agg_hbm,
        src_v, dst_v, bufs, zbuf, agg_sh, *sems):
    gsem = sems[:nbuf]
    ssem = sems[nbuf:]
    c = lax.axis_index("c")
    s = lax.axis_index("s")
    if feature_split:
      pltpu.sync_copy(srcr_hbm.at[s], src_v)
      pltpu.sync_copy(dstr_hbm.at[s], dst_v)
    else:
      wid = c * _NS + s
      pltpu.sync_copy(srcr_hbm.at[wid], src_v)
      pltpu.sync_copy(dstr_hbm.at[wid], dst_v)
    zer = jnp.zeros((16,), jnp.float32)

    def fill_z(i, carry):
      for j in range(width // 16):
        zbuf[i, pl.ds(j * 16, 16)] = zer
      return carry

    lax.fori_loop(0, 125, fill_z, 0)
    for r in range(npt // 125):
      pltpu.sync_copy(zbuf, agg_sh.at[pl.ds(s * npt + r * 125, 125)])
    plsc.subcore_barrier()

    tab = hwp_hbm.at[c] if feature_split else hwp_hbm
    for b in range(nbuf):
      pltpu.async_copy(tab.at[src_v.at[b]], bufs.at[b], gsem[b])

    def rnd(t, carry):
      g0 = t * nbuf
      for b in range(nbuf):
        g = g0 + b
        pltpu.make_async_copy(tab.at[src_v.at[g]], bufs.at[b],
                              gsem[b]).wait()
        pltpu.async_copy(bufs.at[b], agg_sh.at[dst_v.at[g]], ssem[b],
                         add=True)
      for b in range(nbuf):
        g = g0 + b
        nxt = jnp.minimum(g + nbuf, nch - 1)
        pltpu.make_async_copy(bufs.at[b], agg_sh.at[dst_v.at[g]],
                              ssem[b]).wait()
        pltpu.async_copy(tab.at[src_v.at[nxt]], bufs.at[b], gsem[b])
      return carry

    lax.fori_loop(0, nrnd, rnd, 0)
    for b in range(nbuf):
      pltpu.make_async_copy(tab.at[src_v.at[0]], bufs.at[b],
                            gsem[b]).wait()
    plsc.subcore_barrier()
    pltpu.sync_copy(agg_sh.at[pl.ds(s * npt, npt)], agg_hbm.at[c, s])

  return k(hwp_tab, srcr, dstr).reshape(_NC, n, width)


def _tc_head(degp_t, x, w1):
  n = x.shape[0]

  def body(degp_ref, x_ref, w_ref, hwp_ref, dis_ref):
    ones = jnp.ones((degp_ref.shape[0], 1), jnp.float32)
    deg = lax.dot_general(degp_ref[...], ones, (((0,), (0,)), ((), ())),
                          preferred_element_type=jnp.float32) + 1.0
    dis = lax.rsqrt(deg)
    hw = jnp.dot(x_ref[...], w_ref[...],
                 preferred_element_type=jnp.float32) * dis
    hwp_ref[0] = hw[:, 0:64]
    hwp_ref[1] = hw[:, 64:128]
    dis_ref[...] = dis

  return pl.pallas_call(
      body,
      out_shape=(jax.ShapeDtypeStruct((_NC, n, 64), jnp.float32),
                 jax.ShapeDtypeStruct((n, 1), jnp.float32)),
  )(degp_t, x, w1)


def _tc_tail(agg_in, hwp, dis, b, g, be, wn, split_out):
  n = hwp.shape[1]

  def body(agg_ref, hwp_ref, dis_ref, b_ref, g_ref, be_ref, w_ref, out_ref):
    dis = dis_ref[...]
    agg = (agg_ref[...] + hwp_ref[...]) * dis[None] + b_ref[...]
    mu = jnp.mean(agg, axis=1, keepdims=True)
    var = jnp.mean((agg - mu) ** 2, axis=1, keepdims=True)
    h3 = jax.nn.relu((agg - mu) * lax.rsqrt(var + 1e-5) * g_ref[...]
                     + be_ref[...])
    h = jnp.concatenate([h3[0], h3[1]], axis=1)
    hw = jnp.dot(h, w_ref[...], preferred_element_type=jnp.float32) * dis
    if split_out:
      out_ref[0] = hw[:, 0:64]
      out_ref[1] = hw[:, 64:128]
    else:
      out_ref[...] = hw

  out_shape = (jax.ShapeDtypeStruct((_NC, n, 64), jnp.float32) if split_out
               else jax.ShapeDtypeStruct((n, wn.shape[1]), jnp.float32))
  return pl.pallas_call(body, out_shape=out_shape)(
      agg_in, hwp, dis, b.reshape(_NC, 1, 64), g.reshape(_NC, 1, 64),
      be.reshape(_NC, 1, 64), wn)


def _tc_final(aggp, hwp, dis, b3, d_out):
  n = hwp.shape[0]

  def body(aggp_ref, hwp_ref, dis_ref, b_ref, out_ref):
    tot = (aggp_ref[0] + aggp_ref[1] + hwp_ref[...]) * dis_ref[...]
    z = tot[:, 0:d_out] + b_ref[...]
    m = jnp.max(z, axis=1, keepdims=True)
    e = jnp.exp(z - m)
    out_ref[...] = e / jnp.sum(e, axis=1, keepdims=True)

  return pl.pallas_call(
      body,
      out_shape=jax.ShapeDtypeStruct((n, d_out), jnp.float32),
  )(aggp, hwp, dis, b3.reshape(1, -1))


def kernel(x, edge_index, W1, b1, g1, be1, W2, b2, g2, be2, W3, b3):
  n = x.shape[0]
  e = edge_index.shape[1]
  d_out = W3.shape[1]
  assert e % (_NT * _CH) == 0 and n % (_NS * 125) == 0 and n % 16 == 0

  src = edge_index[0]
  dst = edge_index[1]
  srcr_fs = src.reshape(_NS, e // (_NS * _CH), _CH)
  dstr_fs = dst.reshape(_NS, e // (_NS * _CH), _CH)
  srcr_es = src.reshape(_NT, e // (_NT * _CH), _CH)
  dstr_es = dst.reshape(_NT, e // (_NT * _CH), _CH)
  dstr_d = dst.reshape(_NT, e // (_NT * 16), 16)

  degp = _sc_deg(dstr_d, n)
  hw1p, dis = _tc_head(degp, x, W1)
  agg1 = _sc_conv(hw1p, srcr_fs, dstr_fs, n, True, 64, 5)
  hw2p = _tc_tail(agg1, hw1p, dis, b1, g1, be1, W2, True)
  agg2 = _sc_conv(hw2p, srcr_fs, dstr_fs, n, True, 64, 5)
  w3p = jnp.pad(W3, ((0, 0), (0, 48 - d_out)))
  hw3p = _tc_tail(agg2, hw2p, dis, b2, g2, be2, w3p, False)
  agg3p = _sc_conv(hw3p, srcr_es, dstr_es, n, False, 48, 5)
  return _tc_final(agg3p, hw3p, dis, b3, d_out)
